# baseline (speedup 1.0000x reference)
import jax
import jax.numpy as jnp
from jax import lax
from jax.experimental import pallas as pl
from jax.experimental.pallas import tpu as pltpu
from jax.experimental.pallas import tpu_sc as plsc

_IDX_PER_STREAM = 128
_CHUNK = 128
_NBUF = 4


def _make_embed_kernel(batch, grid, vocab, d):
    b_total = batch * grid
    mesh = plsc.VectorSubcoreMesh(core_axis_name="c", subcore_axis_name="s")
    nc, ns = mesh.num_cores, mesh.num_subcores
    nw = nc * ns
    assert _CHUNK == _IDX_PER_STREAM
    assert b_total % (nw * _CHUNK * _NBUF) == 0
    b_per_w = b_total // nw
    n_chunks = b_per_w // _CHUNK
    n_rounds = n_chunks // _NBUF

    def body(idx_hbm, table_hbm, patt_hbm, out_hbm,
             idx_v, rows_v, patt_sh, sem_in, sem_gat, sem_out):
        wid = lax.axis_index("s") * nc + lax.axis_index("c")
        k_base = wid * n_chunks

        def start_in(b, chunk):
            pltpu.async_copy(patt_hbm, rows_v.at[b], sem_in)
            pltpu.async_copy(idx_hbm.at[k_base + chunk], idx_v.at[b], sem_in)

        def wait_in(b):
            pltpu.make_async_copy(patt_hbm, rows_v.at[b], sem_in).wait()
            pltpu.make_async_copy(idx_hbm.at[0], idx_v.at[b], sem_in).wait()

        def start_gather(b):
            pltpu.async_copy(table_hbm.at[idx_v.at[b]], rows_v.at[b],
                             sem_gat, add=True)

        def wait_gather(b):
            pltpu.make_async_copy(table_hbm.at[idx_v.at[b]], rows_v.at[b],
                                  sem_gat).wait()

        def start_out(b, chunk):
            row_off = (k_base + chunk) * _CHUNK
            pltpu.async_copy(rows_v.at[b], out_hbm.at[pl.ds(row_off, _CHUNK)],
                             sem_out)

        def wait_out(b):
            pltpu.make_async_copy(rows_v.at[b],
                                  out_hbm.at[pl.ds(0, _CHUNK)], sem_out).wait()

        for b in range(_NBUF):
            start_in(b, b)

        @pl.loop(0, n_rounds - 1)
        def _round(r):
            k0 = r * _NBUF
            for b in range(_NBUF):
                wait_in(b)
                start_gather(b)
            for b in range(_NBUF):
                wait_gather(b)
                start_out(b, k0 + b)
            for b in range(_NBUF):
                wait_out(b)
                start_in(b, k0 + b + _NBUF)

        k0 = (n_rounds - 1) * _NBUF
        for b in range(_NBUF):
            wait_in(b)
            start_gather(b)
        for b in range(_NBUF):
            wait_gather(b)
            start_out(b, k0 + b)
        for b in range(_NBUF):
            wait_out(b)

    return pl.kernel(
        body,
        out_type=jax.ShapeDtypeStruct((b_total, d), jnp.float32),
        mesh=mesh,
        scratch_types=[
            pltpu.VMEM((_NBUF, _IDX_PER_STREAM), jnp.int32),
            pltpu.VMEM((_NBUF, _CHUNK, d), jnp.float32),
            pltpu.VMEM_SHARED((_CHUNK, d), jnp.float32),
            pltpu.SemaphoreType.DMA,
            pltpu.SemaphoreType.DMA,
            pltpu.SemaphoreType.DMA,
        ],
    )


def kernel(tile_values, value_table, pos_table):
    batch, grid = tile_values.shape
    vocab, d = value_table.shape
    idx = tile_values.astype(jnp.int32).reshape(-1, _IDX_PER_STREAM)
    patt = jnp.tile(pos_table, (_CHUNK // grid, 1))
    k = _make_embed_kernel(batch, grid, vocab, d)
    out = k(idx, value_table, patt)
    return out.reshape(batch, grid, d)


# speedup vs baseline: 1.3881x; 1.3881x over previous
import jax
import jax.numpy as jnp
from jax import lax
from jax.experimental import pallas as pl
from jax.experimental.pallas import tpu as pltpu
from jax.experimental.pallas import tpu_sc as plsc

_IDX_PER_STREAM = 128
_CHUNK = 256
_NBUF = 2


def _make_embed_kernel(batch, grid, vocab, d):
    b_total = batch * grid
    mesh = plsc.VectorSubcoreMesh(core_axis_name="c", subcore_axis_name="s")
    nc, ns = mesh.num_cores, mesh.num_subcores
    nw = nc * ns
    assert _CHUNK % _IDX_PER_STREAM == 0
    n_sub = _CHUNK // _IDX_PER_STREAM
    assert b_total % (nw * _CHUNK * _NBUF) == 0
    b_per_w = b_total // nw
    n_chunks = b_per_w // _CHUNK
    n_rounds = n_chunks // _NBUF

    def body(idx_hbm, table_hbm, patt_hbm, out_hbm,
             idx_v, rows_v, patt_sh, sem_in, sem_gat, sem_out):
        wid = lax.axis_index("s") * nc + lax.axis_index("c")
        k_base = wid * n_chunks

        def start_in(b, chunk):
            pltpu.async_copy(patt_hbm, rows_v.at[b], sem_in)
            pltpu.async_copy(idx_hbm.at[pl.ds((k_base + chunk) * n_sub, n_sub)],
                             idx_v.at[b], sem_in)

        def wait_in(b):
            pltpu.make_async_copy(patt_hbm, rows_v.at[b], sem_in).wait()
            pltpu.make_async_copy(idx_hbm.at[pl.ds(0, n_sub)], idx_v.at[b],
                                  sem_in).wait()

        def start_gather(b):
            for t in range(n_sub):
                pltpu.async_copy(
                    table_hbm.at[idx_v.at[b].at[t]],
                    rows_v.at[b].at[pl.ds(t * _IDX_PER_STREAM, _IDX_PER_STREAM)],
                    sem_gat, add=True)

        def wait_gather(b):
            for t in range(n_sub):
                pltpu.make_async_copy(
                    table_hbm.at[idx_v.at[b].at[t]],
                    rows_v.at[b].at[pl.ds(t * _IDX_PER_STREAM, _IDX_PER_STREAM)],
                    sem_gat).wait()

        def start_out(b, chunk):
            row_off = (k_base + chunk) * _CHUNK
            pltpu.async_copy(rows_v.at[b], out_hbm.at[pl.ds(row_off, _CHUNK)],
                             sem_out)

        def wait_out(b):
            pltpu.make_async_copy(rows_v.at[b],
                                  out_hbm.at[pl.ds(0, _CHUNK)], sem_out).wait()

        for b in range(_NBUF):
            start_in(b, b)

        @pl.loop(0, n_rounds - 1)
        def _round(r):
            k0 = r * _NBUF
            for b in range(_NBUF):
                wait_in(b)
                start_gather(b)
            for b in range(_NBUF):
                wait_gather(b)
                start_out(b, k0 + b)
            for b in range(_NBUF):
                wait_out(b)
                start_in(b, k0 + b + _NBUF)

        k0 = (n_rounds - 1) * _NBUF
        for b in range(_NBUF):
            wait_in(b)
            start_gather(b)
        for b in range(_NBUF):
            wait_gather(b)
            start_out(b, k0 + b)
        for b in range(_NBUF):
            wait_out(b)

    return pl.kernel(
        body,
        out_type=jax.ShapeDtypeStruct((b_total, d), jnp.float32),
        mesh=mesh,
        scratch_types=[
            pltpu.VMEM((_NBUF, n_sub, _IDX_PER_STREAM), jnp.int32),
            pltpu.VMEM((_NBUF, _CHUNK, d), jnp.float32),
            pltpu.VMEM_SHARED((_CHUNK, d), jnp.float32),
            pltpu.SemaphoreType.DMA,
            pltpu.SemaphoreType.DMA,
            pltpu.SemaphoreType.DMA,
        ],
    )


def kernel(tile_values, value_table, pos_table):
    batch, grid = tile_values.shape
    vocab, d = value_table.shape
    idx = tile_values.astype(jnp.int32).reshape(-1, _IDX_PER_STREAM)
    patt = jnp.tile(pos_table, (_CHUNK // grid, 1))
    k = _make_embed_kernel(batch, grid, vocab, d)
    out = k(idx, value_table, patt)
    return out.reshape(batch, grid, d)


# speedup vs baseline: 1.5064x; 1.0852x over previous
import jax
import jax.numpy as jnp
from jax import lax
from jax.experimental import pallas as pl
from jax.experimental.pallas import tpu as pltpu
from jax.experimental.pallas import tpu_sc as plsc

_IDX_PER_STREAM = 512
_CHUNK = 512
_NBUF = 1


def _make_embed_kernel(batch, grid, vocab, d):
    b_total = batch * grid
    mesh = plsc.VectorSubcoreMesh(core_axis_name="c", subcore_axis_name="s")
    nc, ns = mesh.num_cores, mesh.num_subcores
    nw = nc * ns
    assert _CHUNK % _IDX_PER_STREAM == 0
    n_sub = _CHUNK // _IDX_PER_STREAM
    assert b_total % (nw * _CHUNK * _NBUF) == 0
    b_per_w = b_total // nw
    n_chunks = b_per_w // _CHUNK
    n_rounds = n_chunks // _NBUF

    def body(idx_hbm, table_hbm, patt_hbm, out_hbm,
             idx_v, rows_v, sem_in, sem_gat, sem_out):
        wid = lax.axis_index("s") * nc + lax.axis_index("c")
        k_base = wid * n_chunks

        def start_in(b, chunk):
            pltpu.async_copy(patt_hbm, rows_v.at[b], sem_in)
            pltpu.async_copy(idx_hbm.at[pl.ds((k_base + chunk) * _CHUNK, _CHUNK)],
                             idx_v.at[b], sem_in)

        def wait_in(b):
            pltpu.make_async_copy(patt_hbm, rows_v.at[b], sem_in).wait()
            pltpu.make_async_copy(idx_hbm.at[pl.ds(0, _CHUNK)], idx_v.at[b],
                                  sem_in).wait()

        def start_gather(b):
            pltpu.async_copy(table_hbm.at[idx_v.at[b]], rows_v.at[b],
                             sem_gat, add=True)

        def wait_gather(b):
            pltpu.make_async_copy(table_hbm.at[idx_v.at[b]], rows_v.at[b],
                                  sem_gat).wait()

        def start_out(b, chunk):
            row_off = (k_base + chunk) * _CHUNK
            pltpu.async_copy(rows_v.at[b], out_hbm.at[pl.ds(row_off, _CHUNK)],
                             sem_out)

        def wait_out(b):
            pltpu.make_async_copy(rows_v.at[b],
                                  out_hbm.at[pl.ds(0, _CHUNK)], sem_out).wait()

        for b in range(_NBUF):
            start_in(b, b)

        @pl.loop(0, n_rounds - 1)
        def _round(r):
            k0 = r * _NBUF
            for b in range(_NBUF):
                wait_in(b)
                start_gather(b)
            for b in range(_NBUF):
                wait_gather(b)
                start_out(b, k0 + b)
            for b in range(_NBUF):
                wait_out(b)
                start_in(b, k0 + b + _NBUF)

        k0 = (n_rounds - 1) * _NBUF
        for b in range(_NBUF):
            wait_in(b)
            start_gather(b)
        for b in range(_NBUF):
            wait_gather(b)
            start_out(b, k0 + b)
        for b in range(_NBUF):
            wait_out(b)

    return pl.kernel(
        body,
        out_type=jax.ShapeDtypeStruct((b_total, d), jnp.float32),
        mesh=mesh,
        scratch_types=[
            pltpu.VMEM((_NBUF, n_sub * _IDX_PER_STREAM), jnp.int32),
            pltpu.VMEM((_NBUF, _CHUNK, d), jnp.float32),
            pltpu.SemaphoreType.DMA,
            pltpu.SemaphoreType.DMA,
            pltpu.SemaphoreType.DMA,
        ],
    )


def kernel(tile_values, value_table, pos_table):
    batch, grid = tile_values.shape
    vocab, d = value_table.shape
    idx = tile_values.astype(jnp.int32).reshape(-1)
    patt = jnp.tile(pos_table, (_CHUNK // grid, 1))
    k = _make_embed_kernel(batch, grid, vocab, d)
    out = k(idx, value_table, patt)
    return out.reshape(batch, grid, d)


# speedup vs baseline: 2.6961x; 1.7898x over previous
import jax
import jax.numpy as jnp
from jax import lax
from jax.experimental import pallas as pl
from jax.experimental.pallas import tpu as pltpu
from jax.experimental.pallas import tpu_sc as plsc

_LANES = 16
_CHUNK = 256
_NBUF = 2


def _make_embed_kernel(batch, grid, vocab, d):
    b_total = batch * grid
    mesh = plsc.VectorSubcoreMesh(core_axis_name="c", subcore_axis_name="s")
    nc, ns = mesh.num_cores, mesh.num_subcores
    nw = nc * ns
    assert _CHUNK % grid == 0 and d % _LANES == 0
    assert b_total % (nw * _CHUNK * _NBUF) == 0
    b_per_w = b_total // nw
    n_chunks = b_per_w // _CHUNK
    n_rounds = n_chunks // _NBUF

    def body(idx_hbm, table_hbm, pos_hbm, out_hbm,
             idx_v0, idx_v1, rows_v, pos_v, sem_in, sem_gat, sem_out):
        idx_bufs = [idx_v0, idx_v1]
        wid = lax.axis_index("s") * nc + lax.axis_index("c")
        k_base = wid * n_chunks
        pltpu.sync_copy(pos_hbm, pos_v)

        def start_in(b, chunk):
            pltpu.async_copy(idx_hbm.at[pl.ds((k_base + chunk) * _CHUNK, _CHUNK)],
                             idx_bufs[b], sem_in)

        def wait_in(b):
            pltpu.make_async_copy(idx_hbm.at[pl.ds(0, _CHUNK)], idx_bufs[b],
                                  sem_in).wait()

        def start_gather(b):
            pltpu.async_copy(table_hbm.at[idx_bufs[b]], rows_v.at[b], sem_gat)

        def wait_gather(b):
            pltpu.make_async_copy(table_hbm.at[idx_bufs[b]], rows_v.at[b],
                                  sem_gat).wait()

        def add_pos(b):
            rows = rows_v.at[b]
            for j in range(grid):
                pv = [pos_v[j, pl.ds(v * _LANES, _LANES)]
                      for v in range(d // _LANES)]

                @pl.loop(0, _CHUNK // grid)
                def _sweep(g):
                    r = g * grid + j
                    for v in range(d // _LANES):
                        sl = pl.ds(v * _LANES, _LANES)
                        rows[r, sl] = rows[r, sl] + pv[v]

        def start_out(b, chunk):
            row_off = (k_base + chunk) * _CHUNK
            pltpu.async_copy(rows_v.at[b], out_hbm.at[pl.ds(row_off, _CHUNK)],
                             sem_out)

        def wait_out(b):
            pltpu.make_async_copy(rows_v.at[b],
                                  out_hbm.at[pl.ds(0, _CHUNK)], sem_out).wait()

        for b in range(_NBUF):
            start_in(b, b)

        @pl.loop(0, n_rounds - 1)
        def _round(r):
            k0 = r * _NBUF
            for b in range(_NBUF):
                wait_in(b)
                start_gather(b)
            for b in range(_NBUF):
                wait_gather(b)
                add_pos(b)
                start_out(b, k0 + b)
            for b in range(_NBUF):
                wait_out(b)
                start_in(b, k0 + b + _NBUF)

        k0 = (n_rounds - 1) * _NBUF
        for b in range(_NBUF):
            wait_in(b)
            start_gather(b)
        for b in range(_NBUF):
            wait_gather(b)
            add_pos(b)
            start_out(b, k0 + b)
        for b in range(_NBUF):
            wait_out(b)

    return pl.kernel(
        body,
        out_type=jax.ShapeDtypeStruct((b_total, d), jnp.float32),
        mesh=mesh,
        scratch_types=[
            pltpu.VMEM((_CHUNK,), jnp.int32),
            pltpu.VMEM((_CHUNK,), jnp.int32),
            pltpu.VMEM((_NBUF, _CHUNK, d), jnp.float32),
            pltpu.VMEM((grid, d), jnp.float32),
            pltpu.SemaphoreType.DMA,
            pltpu.SemaphoreType.DMA,
            pltpu.SemaphoreType.DMA,
        ],
    )


def kernel(tile_values, value_table, pos_table):
    batch, grid = tile_values.shape
    vocab, d = value_table.shape
    idx = tile_values.astype(jnp.int32).reshape(-1)
    k = _make_embed_kernel(batch, grid, vocab, d)
    out = k(idx, value_table, pos_table)
    return out.reshape(batch, grid, d)
